# baseline (speedup 1.0000x reference)
import functools

import jax
import jax.numpy as jnp
from jax import lax
from jax.experimental import pallas as pl
from jax.experimental.pallas import tpu as pltpu
from jax.experimental.pallas import tpu_sc as plsc

_K = 128
_NW = 32
_LANES = 16


def _splat(ref, j):
    idx = jnp.zeros((_LANES,), jnp.int32) + j
    return plsc.load_gather(ref, [idx])


def _zero_vmem_2d(buf, rows, cols):
    def row(r, c):
        for t in range(cols // _LANES):
            buf[r, pl.ds(t * _LANES, _LANES)] = jnp.zeros((_LANES,), jnp.float32)
        return c
    lax.fori_loop(0, rows, row, 0)


def _rsqrt_bits(s):
    i = lax.bitcast_convert_type(s, jnp.int32)
    y = lax.bitcast_convert_type(jnp.int32(0x5F3759DF) - (i >> 1), jnp.float32)
    for _ in range(3):
        y = y * (1.5 - 0.5 * s * y * y)
    return y


_NH = 5120
_NJ = _K
_TR = _NH + _NJ


def _store_halved_indices(dstb, idxlo, idxhi, jbuf):
    for t in range(_K // _LANES):
        sl = pl.ds(t * _LANES, _LANES)
        dv = dstb[sl]
        jv = jbuf[sl]
        idxlo[sl] = jnp.where(dv < _NH, dv, jv)
        idxhi[sl] = jnp.where(dv >= _NH, dv - _NH, jv)


def _fill_junk_indices(jbuf):
    lanes = lax.iota(jnp.int32, _LANES)
    for t in range(_K // _LANES):
        jbuf[pl.ds(t * _LANES, _LANES)] = lanes + (_NH + t * _LANES)


def _sc_dist_deg(px, py, pz, dst, src, e):
    n = px.shape[0]
    eb = e // _K
    npad = 2 * _NH
    zch = _TR // _K
    och = _NH // _K
    NB = 12
    ngr = (eb + NB - 1) // NB

    mesh = plsc.VectorSubcoreMesh(core_axis_name="c", subcore_axis_name="s",
                                  num_cores=2, num_subcores=16)

    scratch = [
        pltpu.VMEM_SHARED((_TR,), jnp.float32),
        pltpu.VMEM_SHARED((_TR,), jnp.float32),
        pltpu.VMEM((NB * _K,), jnp.int32),
        pltpu.VMEM((NB * _K,), jnp.int32),
        pltpu.VMEM((NB * _K,), jnp.float32),
        pltpu.VMEM((_K,), jnp.int32),
        pltpu.VMEM((_K,), jnp.int32),
        pltpu.VMEM((_K,), jnp.int32),
        pltpu.VMEM((_K,), jnp.int32),
        pltpu.VMEM((_K,), jnp.int32),
        pltpu.VMEM((_K,), jnp.float32),
    ]
    scratch += [pltpu.VMEM((_K,), jnp.float32) for _ in range(12)]
    scratch += [pltpu.SemaphoreType.DMA for _ in range(6)]

    @functools.partial(
        pl.kernel,
        out_type=(
            jax.ShapeDtypeStruct((e + 3200,), jnp.float32),
            jax.ShapeDtypeStruct((2, npad), jnp.float32),
        ),
        mesh=mesh,
        scratch_types=scratch,
    )
    def k(px_h, py_h, pz_h, dst_h, src_h, dist_h, deg_h,
          deg_lo, deg_hi, dsts, srcs, dbs, ixl0, ixl1, ixh0, ixh1,
          jbuf, cb, *rest):
        gb = [rest[0:6], rest[6:12]]
        semG = rest[12:14]
        semL = rest[14:16]
        semH = rest[16:18]
        ixl = [ixl0, ixl1]
        ixh = [ixh0, ixh1]

        cid = lax.axis_index("c")
        sid = lax.axis_index("s")
        wid = sid * 2 + cid

        def zb(t, c):
            cb[pl.ds(t * _LANES, _LANES)] = jnp.zeros((_LANES,), jnp.float32)
            return c
        lax.fori_loop(0, _K // _LANES, zb, 0)

        def zlo(q, c):
            pltpu.sync_copy(cb, deg_lo.at[pl.ds(q * _K, _K)])
            return c
        def zhi(q, c):
            pltpu.sync_copy(cb, deg_hi.at[pl.ds(q * _K, _K)])
            return c
        lax.fori_loop((sid * zch) // 16, ((sid + 1) * zch) // 16, zlo, 0)
        lax.fori_loop((sid * zch) // 16, ((sid + 1) * zch) // 16, zhi, 0)
        plsc.subcore_barrier()

        def ob(t, c):
            cb[pl.ds(t * _LANES, _LANES)] = (
                jnp.zeros((_LANES,), jnp.float32) + 1.0)
            return c
        lax.fori_loop(0, _K // _LANES, ob, 0)
        _fill_junk_indices(jbuf)

        glo = (wid * ngr) // _NW
        ghi = ((wid + 1) * ngr) // _NW

        def granule(g, carry):
            b0 = g * NB
            sbase = b0 * _K
            pltpu.sync_copy(dst_h.at[pl.ds(sbase, NB * _K)], dsts)
            pltpu.sync_copy(src_h.at[pl.ds(sbase, NB * _K)], srcs)

            valid = [b0 + bb < eb for bb in range(NB)]
            dG = [None] * NB
            dL = [None] * NB
            dH = [None] * NB

            def fire_g(bb):
                s = bb & 1
                di = dsts.at[pl.ds(bb * _K, _K)]
                si = srcs.at[pl.ds(bb * _K, _K)]
                dG[bb] = [
                    pltpu.async_copy(px_h.at[di], gb[s][0], semG[s]),
                    pltpu.async_copy(py_h.at[di], gb[s][1], semG[s]),
                    pltpu.async_copy(pz_h.at[di], gb[s][2], semG[s]),
                    pltpu.async_copy(px_h.at[si], gb[s][3], semG[s]),
                    pltpu.async_copy(py_h.at[si], gb[s][4], semG[s]),
                    pltpu.async_copy(pz_h.at[si], gb[s][5], semG[s]),
                ]

            @pl.when(valid[0])
            def _():
                fire_g(0)
            @pl.when(valid[1])
            def _():
                fire_g(1)

            for bb in range(NB):
                s = bb & 1

                @pl.when(valid[bb])
                def _(bb=bb, s=s):
                    for cp in dG[bb]:
                        cp.wait()
                    pxd, pyd, pzd, pxs, pys, pzs = gb[s]
                    for t in range(_K // _LANES):
                        ssl = pl.ds(bb * _K + t * _LANES, _LANES)
                        sl = pl.ds(t * _LANES, _LANES)
                        dx = pxd[sl] - pxs[sl]
                        dy = pyd[sl] - pys[sl]
                        dz = pzd[sl] - pzs[sl]
                        s2 = dx * dx + dy * dy + dz * dz
                        dbs[ssl] = s2 * _rsqrt_bits(jnp.maximum(s2, 1e-30))
                        dv = dsts[ssl]
                        jv = jbuf[sl]
                        ixl[s][sl] = jnp.where(dv < _NH, dv, jv)
                        ixh[s][sl] = jnp.where(dv >= _NH, dv - _NH, jv)
                    dL[bb] = pltpu.async_copy(cb, deg_lo.at[ixl[s]],
                                              semL[s], add=True)
                    dH[bb] = pltpu.async_copy(cb, deg_hi.at[ixh[s]],
                                              semH[s], add=True)

                if bb + 2 < NB:
                    @pl.when(valid[bb])
                    def _(bb=bb):
                        dL[bb].wait()
                        dH[bb].wait()
                    @pl.when(valid[bb + 2])
                    def _(bb=bb):
                        fire_g(bb + 2)
                else:
                    @pl.when(valid[bb])
                    def _(bb=bb):
                        dL[bb].wait()
                        dH[bb].wait()

            pltpu.sync_copy(dbs, dist_h.at[pl.ds(sbase, NB * _K)])
            return carry

        lax.fori_loop(glo, ghi, granule, 0)
        plsc.subcore_barrier()

        def olo(q, c):
            pltpu.sync_copy(deg_lo.at[pl.ds(q * _K, _K)],
                            deg_h.at[cid, pl.ds(q * _K, _K)])
            return c
        def ohi(q, c):
            pltpu.sync_copy(deg_hi.at[pl.ds(q * _K, _K)],
                            deg_h.at[cid, pl.ds(_NH + q * _K, _K)])
            return c
        lax.fori_loop((sid * och) // 16, ((sid + 1) * och) // 16, olo, 0)
        lax.fori_loop((sid * och) // 16, ((sid + 1) * och) // 16, ohi, 0)

    return k(px, py, pz, dst, src)


def _sc_edge(a_tab, b_tab, dst, src, dist, wd, e):
    n, d = a_tab.shape
    eb = e // _K
    npad = 2 * _NH
    zch = _TR // _K
    och = _NH // _K
    NB = 24
    ng = d // _LANES

    mesh = plsc.VectorSubcoreMesh(core_axis_name="c", subcore_axis_name="s",
                                  num_cores=2, num_subcores=16)

    scratch = [
        pltpu.VMEM_SHARED((_TR, d), jnp.float32),
        pltpu.VMEM_SHARED((_TR, d), jnp.float32),
        pltpu.VMEM((_K, d), jnp.float32),
        pltpu.VMEM((_K, d), jnp.float32),
        pltpu.VMEM((NB * _K,), jnp.int32),
        pltpu.VMEM((NB * _K,), jnp.int32),
        pltpu.VMEM((NB * _K + _LANES,), jnp.float32),
        pltpu.VMEM((_K,), jnp.int32),
        pltpu.VMEM((_K,), jnp.int32),
        pltpu.VMEM((_K,), jnp.int32),
        pltpu.VMEM((_K,), jnp.int32),
        pltpu.VMEM((_K,), jnp.int32),
        pltpu.VMEM((d,), jnp.float32),
    ]
    scratch += [pltpu.SemaphoreType.DMA for _ in range(8)]

    @functools.partial(
        pl.kernel,
        out_type=jax.ShapeDtypeStruct((2, npad, d), jnp.float32),
        mesh=mesh,
        scratch_types=scratch,
    )
    def k(a_h, b_h, dst_h, src_h, dist_h, wd_h, out_h,
          s_lo, s_hi, ab0, ab1, dsts, srcs, dbs, ixl0, ixl1, ixh0, ixh1,
          jbuf, wdbuf, *sems):
        ab = [ab0, ab1]
        ixl = [ixl0, ixl1]
        ixh = [ixh0, ixh1]
        semA = sems[0:2]
        semB = sems[2:4]
        semL = sems[4:6]
        semH = sems[6:8]

        cid = lax.axis_index("c")
        sid = lax.axis_index("s")
        wid = sid * 2 + cid

        _zero_vmem_2d(ab0, _K, d)

        def zlo(q, c):
            pltpu.sync_copy(ab0, s_lo.at[pl.ds(q * _K, _K)])
            return c
        def zhi(q, c):
            pltpu.sync_copy(ab0, s_hi.at[pl.ds(q * _K, _K)])
            return c
        lax.fori_loop((sid * zch) // 16, ((sid + 1) * zch) // 16, zlo, 0)
        lax.fori_loop((sid * zch) // 16, ((sid + 1) * zch) // 16, zhi, 0)
        plsc.subcore_barrier()

        pltpu.sync_copy(wd_h, wdbuf)
        wdv = [wdbuf[pl.ds(t * _LANES, _LANES)] for t in range(ng)]
        _fill_junk_indices(jbuf)

        lo = (wid * eb) // _NW
        hi = ((wid + 1) * eb) // _NW
        nst = (hi - lo + NB - 1) // NB

        def stage(sc_i, carry):
            b0 = lo + sc_i * NB
            sbase = b0 * _K
            pltpu.sync_copy(dst_h.at[pl.ds(sbase, NB * _K)], dsts)
            pltpu.sync_copy(src_h.at[pl.ds(sbase, NB * _K)], srcs)
            pltpu.sync_copy(dist_h.at[pl.ds(sbase, NB * _K + _LANES)], dbs)

            valid = [b0 + bb < hi for bb in range(NB)]
            dA = [None] * NB
            dB = [None] * NB
            dL = [None] * NB
            dH = [None] * NB

            def fire_a(bb):
                s = bb & 1
                dA[bb] = pltpu.async_copy(
                    a_h.at[dsts.at[pl.ds(bb * _K, _K)]], ab[s], semA[s])

            def fire_b(bb):
                s = bb & 1
                dA[bb].wait()
                dB[bb] = pltpu.async_copy(
                    b_h.at[srcs.at[pl.ds(bb * _K, _K)]], ab[s], semB[s],
                    add=True)

            @pl.when(valid[0])
            def _():
                fire_a(0)
                fire_b(0)
            @pl.when(valid[1])
            def _():
                fire_a(1)

            for bb in range(NB):
                s = bb & 1
                abr = ab[s]

                @pl.when(valid[bb])
                def _(bb=bb, s=s, abr=abr):
                    dB[bb].wait()
                    for t in range(ng):
                        ssl = pl.ds(bb * _K + t * _LANES, _LANES)
                        sl = pl.ds(t * _LANES, _LANES)
                        dv = dsts[ssl]
                        jv = jbuf[sl]
                        ixl[s][sl] = jnp.where(dv < _NH, dv, jv)
                        ixh[s][sl] = jnp.where(dv >= _NH, dv - _NH, jv)
                    roff = bb * _K

                    def edge(j4, c2):
                        j = j4 * 4
                        for u in range(4):
                            dvv = dbs[pl.ds(roff + j + u, _LANES)]
                            dj = jnp.zeros((_LANES,), jnp.float32) + dvv[0]
                            for t in range(ng):
                                sl = pl.ds(t * _LANES, _LANES)
                                v = abr[j + u, sl] + dj * wdv[t]
                                abr[j + u, sl] = jnp.maximum(v, 0.0)
                        return c2

                    lax.fori_loop(0, _K // 4, edge, 0)
                    dL[bb] = pltpu.async_copy(abr, s_lo.at[ixl[s]],
                                              semL[s], add=True)
                    dH[bb] = pltpu.async_copy(abr, s_hi.at[ixh[s]],
                                              semH[s], add=True)

                if bb + 1 < NB:
                    @pl.when(valid[bb + 1])
                    def _(bb=bb):
                        fire_b(bb + 1)

                @pl.when(valid[bb])
                def _(bb=bb):
                    dL[bb].wait()
                    dH[bb].wait()
                if bb + 2 < NB:
                    @pl.when(valid[bb + 2])
                    def _(bb=bb):
                        fire_a(bb + 2)
            return carry

        lax.fori_loop(0, nst, stage, 0)
        plsc.subcore_barrier()

        def olo(q, c):
            pltpu.sync_copy(s_lo.at[pl.ds(q * _K, _K)],
                            out_h.at[cid, pl.ds(q * _K, _K)])
            return c
        def ohi(q, c):
            pltpu.sync_copy(s_hi.at[pl.ds(q * _K, _K)],
                            out_h.at[cid, pl.ds(_NH + q * _K, _K)])
            return c
        lax.fori_loop((sid * och) // 16, ((sid + 1) * och) // 16, olo, 0)
        lax.fori_loop((sid * och) // 16, ((sid + 1) * och) // 16, ohi, 0)

    return k(a_tab, b_tab, dst, src, dist, wd)


_ROWS = 1000


def _tc_init(x, w_ne, b_ne, wd0, ws0, bm10):
    n, d = x.shape
    grid = (n // _ROWS,)

    def body(x_r, wne_r, bne_r, wd_r, ws_r, bm1_r, h_r, a_r, b_r):
        h = jnp.dot(x_r[...], wne_r[...],
                    preferred_element_type=jnp.float32) + bne_r[...]
        h_r[...] = h
        a_r[...] = jnp.dot(h, wd_r[...],
                           preferred_element_type=jnp.float32) + bm1_r[...]
        b_r[...] = jnp.dot(h, ws_r[...], preferred_element_type=jnp.float32)

    row = pl.BlockSpec((_ROWS, d), lambda m: (m, 0))
    mat = pl.BlockSpec((d, d), lambda m: (0, 0))
    vec = pl.BlockSpec((1, d), lambda m: (0, 0))
    return pl.pallas_call(
        body,
        grid=grid,
        in_specs=[row, mat, vec, mat, mat, vec],
        out_specs=[row, row, row],
        out_shape=[jax.ShapeDtypeStruct((n, d), jnp.float32)] * 3,
        compiler_params=pltpu.CompilerParams(
            dimension_semantics=("parallel",)),
    )(x, w_ne, b_ne.reshape(1, d), wd0, ws0, bm10.reshape(1, d))


def _tc_layer(h, s_parts, deg, wm2, bm2, wu1a, wu1b, bu1, wu2, bu2,
              ln_g, ln_b, residual, wdn, wsn, bm1n):
    n, d = h.shape
    grid = (n // _ROWS,)
    has_next = wdn is not None

    def body(h_r, sp_r, deg_r, wm2_r, bm2_r, wu1a_r, wu1b_r, bu1_r,
             wu2_r, bu2_r, lng_r, lnb_r, *rest):
        if has_next:
            wdn_r, wsn_r, bm1n_r, hn_ref, a_ref, b_ref = rest
        else:
            (hn_ref,) = rest
        h_blk = h_r[...]
        s = sp_r[0] + sp_r[1]
        aggr = (jnp.dot(s, wm2_r[...], preferred_element_type=jnp.float32)
                + deg_r[...] * bm2_r[...])
        t = jnp.dot(h_blk, wu1a_r[...], preferred_element_type=jnp.float32)
        t += jnp.dot(aggr, wu1b_r[...], preferred_element_type=jnp.float32)
        t = jnp.maximum(t + bu1_r[...], 0.0)
        upd = jnp.dot(t, wu2_r[...],
                      preferred_element_type=jnp.float32) + bu2_r[...]
        mu = jnp.mean(upd, axis=1, keepdims=True)
        c = upd - mu
        var = jnp.mean(c * c, axis=1, keepdims=True)
        hn = jnp.maximum(
            c * lax.rsqrt(var + 1e-5) * lng_r[...] + lnb_r[...], 0.0)
        if residual:
            hn = hn + h_blk
        hn_ref[...] = hn
        if has_next:
            a_ref[...] = jnp.dot(hn, wdn_r[...],
                                 preferred_element_type=jnp.float32) + bm1n_r[...]
            b_ref[...] = jnp.dot(hn, wsn_r[...],
                                 preferred_element_type=jnp.float32)

    row = pl.BlockSpec((_ROWS, d), lambda m: (m, 0))
    rows2 = pl.BlockSpec((2, _ROWS, d), lambda m: (0, m, 0))
    col = pl.BlockSpec((_ROWS, 1), lambda m: (m, 0))
    mat = pl.BlockSpec((d, d), lambda m: (0, 0))
    vec = pl.BlockSpec((1, d), lambda m: (0, 0))

    in_specs = [row, rows2, col, mat, vec, mat, mat, vec, mat, vec, vec, vec]
    args = [h, s_parts, deg, wm2, bm2.reshape(1, d), wu1a, wu1b,
            bu1.reshape(1, d), wu2, bu2.reshape(1, d),
            ln_g.reshape(1, d), ln_b.reshape(1, d)]
    out_specs = [row]
    out_shape = [jax.ShapeDtypeStruct((n, d), jnp.float32)]
    if has_next:
        in_specs += [mat, mat, vec]
        args += [wdn, wsn, bm1n.reshape(1, d)]
        out_specs += [row, row]
        out_shape += [jax.ShapeDtypeStruct((n, d), jnp.float32)] * 2

    return pl.pallas_call(
        body,
        grid=grid,
        in_specs=in_specs,
        out_specs=out_specs,
        out_shape=out_shape,
        compiler_params=pltpu.CompilerParams(
            dimension_semantics=("parallel",)),
    )(*args)


def _tc_pool_mlp(h, batch3, g, w1, b1, w2, b2, w3r):
    n, d = h.shape
    grid = (n // _ROWS,)
    nb = n // _ROWS
    d2 = w2.shape[1]

    def body(h_r, bat_r, w1_r, b1_r, w2_r, b2_r, w3_r, out_r, pool, cnt):
        m = pl.program_id(0)

        @pl.when(m == 0)
        def _():
            pool[...] = jnp.zeros_like(pool)
            cnt[...] = jnp.zeros_like(cnt)

        bat = bat_r[0, 0, :]
        gid = lax.broadcasted_iota(jnp.int32, (g, _ROWS), 0)
        oh = (bat[None, :] == gid).astype(jnp.float32)
        pool[...] += jnp.dot(oh, h_r[...], preferred_element_type=jnp.float32)
        cnt[...] += jnp.dot(oh, jnp.ones((_ROWS, d), jnp.float32),
                            preferred_element_type=jnp.float32)

        @pl.when(m == nb - 1)
        def _():
            pooled = pool[...] / jnp.maximum(cnt[...], 1.0)
            o = jnp.maximum(
                jnp.dot(pooled, w1_r[...],
                        preferred_element_type=jnp.float32) + b1_r[...], 0.0)
            o = jnp.maximum(
                jnp.dot(o, w2_r[...],
                        preferred_element_type=jnp.float32) + b2_r[...], 0.0)
            out_r[...] = jnp.sum(o * w3_r[...], axis=1, keepdims=True)

    return pl.pallas_call(
        body,
        grid=grid,
        in_specs=[
            pl.BlockSpec((_ROWS, d), lambda m: (m, 0)),
            pl.BlockSpec((1, 1, _ROWS), lambda m: (m, 0, 0)),
            pl.BlockSpec((d, d), lambda m: (0, 0)),
            pl.BlockSpec((1, d), lambda m: (0, 0)),
            pl.BlockSpec((d, d2), lambda m: (0, 0)),
            pl.BlockSpec((1, d2), lambda m: (0, 0)),
            pl.BlockSpec((1, d2), lambda m: (0, 0)),
        ],
        out_specs=pl.BlockSpec((g, 1), lambda m: (0, 0)),
        out_shape=jax.ShapeDtypeStruct((g, 1), jnp.float32),
        scratch_shapes=[
            pltpu.VMEM((g, d), jnp.float32),
            pltpu.VMEM((g, d), jnp.float32),
        ],
        compiler_params=pltpu.CompilerParams(
            dimension_semantics=("arbitrary",)),
    )(h, batch3, w1, b1.reshape(1, d), w2, b2.reshape(1, d2), w3r)


def kernel(x, edge_index, edge_attr, pos, batch, W_ne, b_ne, W_ee, b_ee,
           Wm1, bm1, Wm2, bm2, Wu1, bu1, Wu2, bu2, ln_g, ln_b,
           Wmlp1, bmlp1, Wmlp2, bmlp2, Wmlp3, bmlp3):
    n, d = x.shape
    num_layers = Wm1.shape[0]
    g = 64

    dst = edge_index[1]
    src = edge_index[0]
    e = edge_index.shape[1]
    pad_i = jnp.zeros((3072,), jnp.int32)
    dst_p = jnp.concatenate([dst, pad_i])
    src_p = jnp.concatenate([src, pad_i])
    px = pos[:, 0]
    py = pos[:, 1]
    pz = pos[:, 2]

    dist, deg2 = _sc_dist_deg(px, py, pz, dst_p, src_p, e)
    deg = (deg2[0, :n] + deg2[1, :n]).reshape(n, 1)

    wd_all = [Wm1[i, :d, :] for i in range(num_layers)]
    ws_all = [Wm1[i, d:2 * d, :] for i in range(num_layers)]
    wdist_all = [Wm1[i, 2 * d, :] for i in range(num_layers)]

    h, a_tab, b_tab = _tc_init(x, W_ne, b_ne, wd_all[0], ws_all[0], bm1[0])

    for i in range(num_layers):
        s_parts = _sc_edge(a_tab, b_tab, dst_p, src_p, dist,
                           wdist_all[i], e)
        nxt = i + 1
        if nxt < num_layers:
            h, a_tab, b_tab = _tc_layer(
                h, s_parts, deg, Wm2[i], bm2[i],
                Wu1[i, :d, :], Wu1[i, d:, :], bu1[i], Wu2[i], bu2[i],
                ln_g[i], ln_b[i], residual=(i > 0),
                wdn=wd_all[nxt], wsn=ws_all[nxt], bm1n=bm1[nxt])
        else:
            (h,) = _tc_layer(
                h, s_parts, deg, Wm2[i], bm2[i],
                Wu1[i, :d, :], Wu1[i, d:, :], bu1[i], Wu2[i], bu2[i],
                ln_g[i], ln_b[i], residual=(i > 0),
                wdn=None, wsn=None, bm1n=None)

    batch3 = batch.reshape(n // _ROWS, 1, _ROWS)
    out = _tc_pool_mlp(h, batch3, g, Wmlp1, bmlp1, Wmlp2, bmlp2,
                       Wmlp3.reshape(1, d // 2))
    return out + bmlp3.reshape(1, 1)


# speedup vs baseline: 1.0345x; 1.0345x over previous
import functools

import jax
import jax.numpy as jnp
from jax import lax
from jax.experimental import pallas as pl
from jax.experimental.pallas import tpu as pltpu
from jax.experimental.pallas import tpu_sc as plsc

_K = 128
_NW = 32
_LANES = 16


def _splat(ref, j):
    idx = jnp.zeros((_LANES,), jnp.int32) + j
    return plsc.load_gather(ref, [idx])


def _zero_vmem_2d(buf, rows, cols):
    def row(r, c):
        for t in range(cols // _LANES):
            buf[r, pl.ds(t * _LANES, _LANES)] = jnp.zeros((_LANES,), jnp.float32)
        return c
    lax.fori_loop(0, rows, row, 0)


def _rsqrt_bits(s):
    i = lax.bitcast_convert_type(s, jnp.int32)
    y = lax.bitcast_convert_type(jnp.int32(0x5F3759DF) - (i >> 1), jnp.float32)
    for _ in range(3):
        y = y * (1.5 - 0.5 * s * y * y)
    return y


_NH = 5120
_NJ = _K
_TR = _NH + _NJ


def _store_halved_indices(dstb, idxlo, idxhi, jbuf):
    for t in range(_K // _LANES):
        sl = pl.ds(t * _LANES, _LANES)
        dv = dstb[sl]
        jv = jbuf[sl]
        idxlo[sl] = jnp.where(dv < _NH, dv, jv)
        idxhi[sl] = jnp.where(dv >= _NH, dv - _NH, jv)


def _fill_junk_indices(jbuf):
    lanes = lax.iota(jnp.int32, _LANES)
    for t in range(_K // _LANES):
        jbuf[pl.ds(t * _LANES, _LANES)] = lanes + (_NH + t * _LANES)


def _sc_dist_deg(px, py, pz, dst, src, e):
    n = px.shape[0]
    eb = e // _K
    npad = 2 * _NH
    zch = _TR // _K
    och = _NH // _K
    NB = 12
    ngr = (eb + NB - 1) // NB

    mesh = plsc.VectorSubcoreMesh(core_axis_name="c", subcore_axis_name="s",
                                  num_cores=2, num_subcores=16)

    scratch = [
        pltpu.VMEM_SHARED((_TR,), jnp.float32),
        pltpu.VMEM_SHARED((_TR,), jnp.float32),
        pltpu.VMEM((NB * _K,), jnp.int32),
        pltpu.VMEM((NB * _K,), jnp.int32),
        pltpu.VMEM((NB * _K,), jnp.float32),
        pltpu.VMEM((_K,), jnp.int32),
        pltpu.VMEM((_K,), jnp.int32),
        pltpu.VMEM((_K,), jnp.int32),
        pltpu.VMEM((_K,), jnp.int32),
        pltpu.VMEM((_K,), jnp.int32),
        pltpu.VMEM((_K,), jnp.float32),
    ]
    scratch += [pltpu.VMEM((_K,), jnp.float32) for _ in range(12)]
    scratch += [pltpu.SemaphoreType.DMA for _ in range(6)]

    @functools.partial(
        pl.kernel,
        out_type=(
            jax.ShapeDtypeStruct((e + 3200,), jnp.float32),
            jax.ShapeDtypeStruct((2, npad), jnp.float32),
        ),
        mesh=mesh,
        scratch_types=scratch,
    )
    def k(px_h, py_h, pz_h, dst_h, src_h, dist_h, deg_h,
          deg_lo, deg_hi, dsts, srcs, dbs, ixl0, ixl1, ixh0, ixh1,
          jbuf, cb, *rest):
        gb = [rest[0:6], rest[6:12]]
        semG = rest[12:14]
        semL = rest[14:16]
        semH = rest[16:18]
        ixl = [ixl0, ixl1]
        ixh = [ixh0, ixh1]

        cid = lax.axis_index("c")
        sid = lax.axis_index("s")
        wid = sid * 2 + cid

        def zb(t, c):
            cb[pl.ds(t * _LANES, _LANES)] = jnp.zeros((_LANES,), jnp.float32)
            return c
        lax.fori_loop(0, _K // _LANES, zb, 0)

        def zlo(q, c):
            pltpu.sync_copy(cb, deg_lo.at[pl.ds(q * _K, _K)])
            return c
        def zhi(q, c):
            pltpu.sync_copy(cb, deg_hi.at[pl.ds(q * _K, _K)])
            return c
        lax.fori_loop((sid * zch) // 16, ((sid + 1) * zch) // 16, zlo, 0)
        lax.fori_loop((sid * zch) // 16, ((sid + 1) * zch) // 16, zhi, 0)
        plsc.subcore_barrier()

        def ob(t, c):
            cb[pl.ds(t * _LANES, _LANES)] = (
                jnp.zeros((_LANES,), jnp.float32) + 1.0)
            return c
        lax.fori_loop(0, _K // _LANES, ob, 0)
        _fill_junk_indices(jbuf)

        glo = (wid * ngr) // _NW
        ghi = ((wid + 1) * ngr) // _NW

        def granule(g, carry):
            b0 = g * NB
            sbase = b0 * _K
            pltpu.sync_copy(dst_h.at[pl.ds(sbase, NB * _K)], dsts)
            pltpu.sync_copy(src_h.at[pl.ds(sbase, NB * _K)], srcs)

            valid = [b0 + bb < eb for bb in range(NB)]
            dG = [None] * NB
            dL = [None] * NB
            dH = [None] * NB

            def fire_g(bb):
                s = bb & 1
                di = dsts.at[pl.ds(bb * _K, _K)]
                si = srcs.at[pl.ds(bb * _K, _K)]
                dG[bb] = [
                    pltpu.async_copy(px_h.at[di], gb[s][0], semG[s]),
                    pltpu.async_copy(py_h.at[di], gb[s][1], semG[s]),
                    pltpu.async_copy(pz_h.at[di], gb[s][2], semG[s]),
                    pltpu.async_copy(px_h.at[si], gb[s][3], semG[s]),
                    pltpu.async_copy(py_h.at[si], gb[s][4], semG[s]),
                    pltpu.async_copy(pz_h.at[si], gb[s][5], semG[s]),
                ]

            @pl.when(valid[0])
            def _():
                fire_g(0)
            @pl.when(valid[1])
            def _():
                fire_g(1)

            for bb in range(NB):
                s = bb & 1

                @pl.when(valid[bb])
                def _(bb=bb, s=s):
                    for cp in dG[bb]:
                        cp.wait()
                    pxd, pyd, pzd, pxs, pys, pzs = gb[s]
                    for t in range(_K // _LANES):
                        ssl = pl.ds(bb * _K + t * _LANES, _LANES)
                        sl = pl.ds(t * _LANES, _LANES)
                        dx = pxd[sl] - pxs[sl]
                        dy = pyd[sl] - pys[sl]
                        dz = pzd[sl] - pzs[sl]
                        s2 = dx * dx + dy * dy + dz * dz
                        dbs[ssl] = s2 * _rsqrt_bits(jnp.maximum(s2, 1e-30))
                        dv = dsts[ssl]
                        jv = jbuf[sl]
                        ixl[s][sl] = jnp.where(dv < _NH, dv, jv)
                        ixh[s][sl] = jnp.where(dv >= _NH, dv - _NH, jv)
                    dL[bb] = pltpu.async_copy(cb, deg_lo.at[ixl[s]],
                                              semL[s], add=True)
                    dH[bb] = pltpu.async_copy(cb, deg_hi.at[ixh[s]],
                                              semH[s], add=True)

                if bb + 2 < NB:
                    @pl.when(valid[bb])
                    def _(bb=bb):
                        dL[bb].wait()
                        dH[bb].wait()
                    @pl.when(valid[bb + 2])
                    def _(bb=bb):
                        fire_g(bb + 2)
                else:
                    @pl.when(valid[bb])
                    def _(bb=bb):
                        dL[bb].wait()
                        dH[bb].wait()

            pltpu.sync_copy(dbs, dist_h.at[pl.ds(sbase, NB * _K)])
            return carry

        lax.fori_loop(glo, ghi, granule, 0)
        plsc.subcore_barrier()

        def olo(q, c):
            pltpu.sync_copy(deg_lo.at[pl.ds(q * _K, _K)],
                            deg_h.at[cid, pl.ds(q * _K, _K)])
            return c
        def ohi(q, c):
            pltpu.sync_copy(deg_hi.at[pl.ds(q * _K, _K)],
                            deg_h.at[cid, pl.ds(_NH + q * _K, _K)])
            return c
        lax.fori_loop((sid * och) // 16, ((sid + 1) * och) // 16, olo, 0)
        lax.fori_loop((sid * och) // 16, ((sid + 1) * och) // 16, ohi, 0)

    return k(px, py, pz, dst, src)


def _sc_edge(a_tab, b_tab, dst, src, dist, wd, e):
    n, d = a_tab.shape
    eb = e // _K
    npad = 2 * _NH
    zch = _TR // _K
    och = _NH // _K
    NB = 32
    ng = d // _LANES

    mesh = plsc.VectorSubcoreMesh(core_axis_name="c", subcore_axis_name="s",
                                  num_cores=2, num_subcores=16)

    scratch = [
        pltpu.VMEM_SHARED((_TR, d), jnp.float32),
        pltpu.VMEM_SHARED((_TR, d), jnp.float32),
        pltpu.VMEM((_K, d), jnp.float32),
        pltpu.VMEM((_K, d), jnp.float32),
        pltpu.VMEM((NB * _K,), jnp.int32),
        pltpu.VMEM((NB * _K,), jnp.int32),
        pltpu.VMEM((NB * _K + _LANES,), jnp.float32),
        pltpu.VMEM((_K,), jnp.int32),
        pltpu.VMEM((_K,), jnp.int32),
        pltpu.VMEM((_K,), jnp.int32),
        pltpu.VMEM((_K,), jnp.int32),
        pltpu.VMEM((_K,), jnp.int32),
        pltpu.VMEM((d,), jnp.float32),
    ]
    scratch += [pltpu.SemaphoreType.DMA for _ in range(8)]

    @functools.partial(
        pl.kernel,
        out_type=jax.ShapeDtypeStruct((2, npad, d), jnp.float32),
        mesh=mesh,
        scratch_types=scratch,
    )
    def k(a_h, b_h, dst_h, src_h, dist_h, wd_h, out_h,
          s_lo, s_hi, ab0, ab1, dsts, srcs, dbs, ixl0, ixl1, ixh0, ixh1,
          jbuf, wdbuf, *sems):
        ab = [ab0, ab1]
        ixl = [ixl0, ixl1]
        ixh = [ixh0, ixh1]
        semA = sems[0:2]
        semB = sems[2:4]
        semL = sems[4:6]
        semH = sems[6:8]

        cid = lax.axis_index("c")
        sid = lax.axis_index("s")
        wid = sid * 2 + cid

        _zero_vmem_2d(ab0, _K, d)

        def zlo(q, c):
            pltpu.sync_copy(ab0, s_lo.at[pl.ds(q * _K, _K)])
            return c
        def zhi(q, c):
            pltpu.sync_copy(ab0, s_hi.at[pl.ds(q * _K, _K)])
            return c
        lax.fori_loop((sid * zch) // 16, ((sid + 1) * zch) // 16, zlo, 0)
        lax.fori_loop((sid * zch) // 16, ((sid + 1) * zch) // 16, zhi, 0)
        plsc.subcore_barrier()

        pltpu.sync_copy(wd_h, wdbuf)
        wdv = [wdbuf[pl.ds(t * _LANES, _LANES)] for t in range(ng)]
        _fill_junk_indices(jbuf)

        lo = (wid * eb) // _NW
        hi = ((wid + 1) * eb) // _NW
        nst = (hi - lo + NB - 1) // NB

        def stage(sc_i, carry):
            b0 = lo + sc_i * NB
            sbase = b0 * _K
            pltpu.sync_copy(dst_h.at[pl.ds(sbase, NB * _K)], dsts)
            pltpu.sync_copy(src_h.at[pl.ds(sbase, NB * _K)], srcs)
            pltpu.sync_copy(dist_h.at[pl.ds(sbase, NB * _K + _LANES)], dbs)

            valid = [b0 + bb < hi for bb in range(NB)]
            dA = [None] * NB
            dB = [None] * NB
            dL = [None] * NB
            dH = [None] * NB

            def fire_a(bb):
                s = bb & 1
                dA[bb] = pltpu.async_copy(
                    a_h.at[dsts.at[pl.ds(bb * _K, _K)]], ab[s], semA[s])

            def fire_b(bb):
                s = bb & 1
                dA[bb].wait()
                dB[bb] = pltpu.async_copy(
                    b_h.at[srcs.at[pl.ds(bb * _K, _K)]], ab[s], semB[s],
                    add=True)

            @pl.when(valid[0])
            def _():
                fire_a(0)
                fire_b(0)
            @pl.when(valid[1])
            def _():
                fire_a(1)

            for bb in range(NB):
                s = bb & 1
                abr = ab[s]

                @pl.when(valid[bb])
                def _(bb=bb, s=s, abr=abr):
                    for t in range(ng):
                        ssl = pl.ds(bb * _K + t * _LANES, _LANES)
                        sl = pl.ds(t * _LANES, _LANES)
                        dv = dsts[ssl]
                        jv = jbuf[sl]
                        ixl[s][sl] = jnp.where(dv < _NH, dv, jv)
                        ixh[s][sl] = jnp.where(dv >= _NH, dv - _NH, jv)
                    dB[bb].wait()
                    roff = bb * _K

                    def edge(j, c2):
                        dvv = dbs[pl.ds(roff + j, _LANES)]
                        dj = jnp.zeros((_LANES,), jnp.float32) + dvv[0]
                        for t in range(ng):
                            sl = pl.ds(t * _LANES, _LANES)
                            v = abr[j, sl] + dj * wdv[t]
                            abr[j, sl] = jnp.maximum(v, 0.0)
                        return c2

                    lax.fori_loop(0, _K, edge, 0)
                    dL[bb] = pltpu.async_copy(abr, s_lo.at[ixl[s]],
                                              semL[s], add=True)
                    dH[bb] = pltpu.async_copy(abr, s_hi.at[ixh[s]],
                                              semH[s], add=True)

                if bb + 1 < NB:
                    @pl.when(valid[bb + 1])
                    def _(bb=bb):
                        fire_b(bb + 1)

                @pl.when(valid[bb])
                def _(bb=bb):
                    dL[bb].wait()
                    dH[bb].wait()
                if bb + 2 < NB:
                    @pl.when(valid[bb + 2])
                    def _(bb=bb):
                        fire_a(bb + 2)
            return carry

        lax.fori_loop(0, nst, stage, 0)
        plsc.subcore_barrier()

        def olo(q, c):
            pltpu.sync_copy(s_lo.at[pl.ds(q * _K, _K)],
                            out_h.at[cid, pl.ds(q * _K, _K)])
            return c
        def ohi(q, c):
            pltpu.sync_copy(s_hi.at[pl.ds(q * _K, _K)],
                            out_h.at[cid, pl.ds(_NH + q * _K, _K)])
            return c
        lax.fori_loop((sid * och) // 16, ((sid + 1) * och) // 16, olo, 0)
        lax.fori_loop((sid * och) // 16, ((sid + 1) * och) // 16, ohi, 0)

    return k(a_tab, b_tab, dst, src, dist, wd)


_ROWS = 1000


def _tc_init(x, w_ne, b_ne, wd0, ws0, bm10):
    n, d = x.shape
    grid = (n // _ROWS,)

    def body(x_r, wne_r, bne_r, wd_r, ws_r, bm1_r, h_r, a_r, b_r):
        h = jnp.dot(x_r[...], wne_r[...],
                    preferred_element_type=jnp.float32) + bne_r[...]
        h_r[...] = h
        a_r[...] = jnp.dot(h, wd_r[...],
                           preferred_element_type=jnp.float32) + bm1_r[...]
        b_r[...] = jnp.dot(h, ws_r[...], preferred_element_type=jnp.float32)

    row = pl.BlockSpec((_ROWS, d), lambda m: (m, 0))
    mat = pl.BlockSpec((d, d), lambda m: (0, 0))
    vec = pl.BlockSpec((1, d), lambda m: (0, 0))
    return pl.pallas_call(
        body,
        grid=grid,
        in_specs=[row, mat, vec, mat, mat, vec],
        out_specs=[row, row, row],
        out_shape=[jax.ShapeDtypeStruct((n, d), jnp.float32)] * 3,
        compiler_params=pltpu.CompilerParams(
            dimension_semantics=("parallel",)),
    )(x, w_ne, b_ne.reshape(1, d), wd0, ws0, bm10.reshape(1, d))


def _tc_layer(h, s_parts, deg, wm2, bm2, wu1a, wu1b, bu1, wu2, bu2,
              ln_g, ln_b, residual, wdn, wsn, bm1n):
    n, d = h.shape
    grid = (n // _ROWS,)
    has_next = wdn is not None

    def body(h_r, sp_r, deg_r, wm2_r, bm2_r, wu1a_r, wu1b_r, bu1_r,
             wu2_r, bu2_r, lng_r, lnb_r, *rest):
        if has_next:
            wdn_r, wsn_r, bm1n_r, hn_ref, a_ref, b_ref = rest
        else:
            (hn_ref,) = rest
        h_blk = h_r[...]
        s = sp_r[0] + sp_r[1]
        aggr = (jnp.dot(s, wm2_r[...], preferred_element_type=jnp.float32)
                + deg_r[...] * bm2_r[...])
        t = jnp.dot(h_blk, wu1a_r[...], preferred_element_type=jnp.float32)
        t += jnp.dot(aggr, wu1b_r[...], preferred_element_type=jnp.float32)
        t = jnp.maximum(t + bu1_r[...], 0.0)
        upd = jnp.dot(t, wu2_r[...],
                      preferred_element_type=jnp.float32) + bu2_r[...]
        mu = jnp.mean(upd, axis=1, keepdims=True)
        c = upd - mu
        var = jnp.mean(c * c, axis=1, keepdims=True)
        hn = jnp.maximum(
            c * lax.rsqrt(var + 1e-5) * lng_r[...] + lnb_r[...], 0.0)
        if residual:
            hn = hn + h_blk
        hn_ref[...] = hn
        if has_next:
            a_ref[...] = jnp.dot(hn, wdn_r[...],
                                 preferred_element_type=jnp.float32) + bm1n_r[...]
            b_ref[...] = jnp.dot(hn, wsn_r[...],
                                 preferred_element_type=jnp.float32)

    row = pl.BlockSpec((_ROWS, d), lambda m: (m, 0))
    rows2 = pl.BlockSpec((2, _ROWS, d), lambda m: (0, m, 0))
    col = pl.BlockSpec((_ROWS, 1), lambda m: (m, 0))
    mat = pl.BlockSpec((d, d), lambda m: (0, 0))
    vec = pl.BlockSpec((1, d), lambda m: (0, 0))

    in_specs = [row, rows2, col, mat, vec, mat, mat, vec, mat, vec, vec, vec]
    args = [h, s_parts, deg, wm2, bm2.reshape(1, d), wu1a, wu1b,
            bu1.reshape(1, d), wu2, bu2.reshape(1, d),
            ln_g.reshape(1, d), ln_b.reshape(1, d)]
    out_specs = [row]
    out_shape = [jax.ShapeDtypeStruct((n, d), jnp.float32)]
    if has_next:
        in_specs += [mat, mat, vec]
        args += [wdn, wsn, bm1n.reshape(1, d)]
        out_specs += [row, row]
        out_shape += [jax.ShapeDtypeStruct((n, d), jnp.float32)] * 2

    return pl.pallas_call(
        body,
        grid=grid,
        in_specs=in_specs,
        out_specs=out_specs,
        out_shape=out_shape,
        compiler_params=pltpu.CompilerParams(
            dimension_semantics=("parallel",)),
    )(*args)


def _tc_pool_mlp(h, batch3, g, w1, b1, w2, b2, w3r):
    n, d = h.shape
    grid = (n // _ROWS,)
    nb = n // _ROWS
    d2 = w2.shape[1]

    def body(h_r, bat_r, w1_r, b1_r, w2_r, b2_r, w3_r, out_r, pool, cnt):
        m = pl.program_id(0)

        @pl.when(m == 0)
        def _():
            pool[...] = jnp.zeros_like(pool)
            cnt[...] = jnp.zeros_like(cnt)

        bat = bat_r[0, 0, :]
        gid = lax.broadcasted_iota(jnp.int32, (g, _ROWS), 0)
        oh = (bat[None, :] == gid).astype(jnp.float32)
        pool[...] += jnp.dot(oh, h_r[...], preferred_element_type=jnp.float32)
        cnt[...] += jnp.dot(oh, jnp.ones((_ROWS, d), jnp.float32),
                            preferred_element_type=jnp.float32)

        @pl.when(m == nb - 1)
        def _():
            pooled = pool[...] / jnp.maximum(cnt[...], 1.0)
            o = jnp.maximum(
                jnp.dot(pooled, w1_r[...],
                        preferred_element_type=jnp.float32) + b1_r[...], 0.0)
            o = jnp.maximum(
                jnp.dot(o, w2_r[...],
                        preferred_element_type=jnp.float32) + b2_r[...], 0.0)
            out_r[...] = jnp.sum(o * w3_r[...], axis=1, keepdims=True)

    return pl.pallas_call(
        body,
        grid=grid,
        in_specs=[
            pl.BlockSpec((_ROWS, d), lambda m: (m, 0)),
            pl.BlockSpec((1, 1, _ROWS), lambda m: (m, 0, 0)),
            pl.BlockSpec((d, d), lambda m: (0, 0)),
            pl.BlockSpec((1, d), lambda m: (0, 0)),
            pl.BlockSpec((d, d2), lambda m: (0, 0)),
            pl.BlockSpec((1, d2), lambda m: (0, 0)),
            pl.BlockSpec((1, d2), lambda m: (0, 0)),
        ],
        out_specs=pl.BlockSpec((g, 1), lambda m: (0, 0)),
        out_shape=jax.ShapeDtypeStruct((g, 1), jnp.float32),
        scratch_shapes=[
            pltpu.VMEM((g, d), jnp.float32),
            pltpu.VMEM((g, d), jnp.float32),
        ],
        compiler_params=pltpu.CompilerParams(
            dimension_semantics=("arbitrary",)),
    )(h, batch3, w1, b1.reshape(1, d), w2, b2.reshape(1, d2), w3r)


def kernel(x, edge_index, edge_attr, pos, batch, W_ne, b_ne, W_ee, b_ee,
           Wm1, bm1, Wm2, bm2, Wu1, bu1, Wu2, bu2, ln_g, ln_b,
           Wmlp1, bmlp1, Wmlp2, bmlp2, Wmlp3, bmlp3):
    n, d = x.shape
    num_layers = Wm1.shape[0]
    g = 64

    dst = edge_index[1]
    src = edge_index[0]
    e = edge_index.shape[1]
    pad_i = jnp.zeros((3072,), jnp.int32)
    dst_p = jnp.concatenate([dst, pad_i])
    src_p = jnp.concatenate([src, pad_i])
    px = pos[:, 0]
    py = pos[:, 1]
    pz = pos[:, 2]

    dist, deg2 = _sc_dist_deg(px, py, pz, dst_p, src_p, e)
    deg = (deg2[0, :n] + deg2[1, :n]).reshape(n, 1)

    wd_all = [Wm1[i, :d, :] for i in range(num_layers)]
    ws_all = [Wm1[i, d:2 * d, :] for i in range(num_layers)]
    wdist_all = [Wm1[i, 2 * d, :] for i in range(num_layers)]

    h, a_tab, b_tab = _tc_init(x, W_ne, b_ne, wd_all[0], ws_all[0], bm1[0])

    for i in range(num_layers):
        s_parts = _sc_edge(a_tab, b_tab, dst_p, src_p, dist,
                           wdist_all[i], e)
        nxt = i + 1
        if nxt < num_layers:
            h, a_tab, b_tab = _tc_layer(
                h, s_parts, deg, Wm2[i], bm2[i],
                Wu1[i, :d, :], Wu1[i, d:, :], bu1[i], Wu2[i], bu2[i],
                ln_g[i], ln_b[i], residual=(i > 0),
                wdn=wd_all[nxt], wsn=ws_all[nxt], bm1n=bm1[nxt])
        else:
            (h,) = _tc_layer(
                h, s_parts, deg, Wm2[i], bm2[i],
                Wu1[i, :d, :], Wu1[i, d:, :], bu1[i], Wu2[i], bu2[i],
                ln_g[i], ln_b[i], residual=(i > 0),
                wdn=None, wsn=None, bm1n=None)

    batch3 = batch.reshape(n // _ROWS, 1, _ROWS)
    out = _tc_pool_mlp(h, batch3, g, Wmlp1, bmlp1, Wmlp2, bmlp2,
                       Wmlp3.reshape(1, d // 2))
    return out + bmlp3.reshape(1, 1)
